# baseline (speedup 1.0000x reference)
import jax
import jax.numpy as jnp
from jax.experimental import pallas as pl
from jax.experimental.pallas import tpu as pltpu

_N, _C, _M = 1024, 81, 28
_K = 8
_STEPS = _N // _K


def _body(labels_ref, *refs):
    logits_refs = refs[:_K]
    targets_ref = refs[_K]
    out_ref = refs[_K + 1]
    i = pl.program_id(0)
    t = targets_ref[...]
    x = jnp.concatenate([r[0] for r in logits_refs], axis=0)
    bce = jnp.maximum(x, 0.0) - x * t + jnp.log1p(jnp.exp(-jnp.abs(x)))
    s = jnp.sum(bce)
    prev = jnp.where(i == 0, 0.0, out_ref[0, 0])
    tot = prev + s
    out_ref[0, 0] = jnp.where(
        i == pl.num_programs(0) - 1, tot / (_N * _M * _M), tot
    )


def kernel(mask_logits, labels, mask_targets):
    labels_i32 = labels.astype(jnp.int32)

    def _logit_spec(j):
        return pl.BlockSpec(
            (1, 1, _M, _M),
            lambda i, lbl, j=j: (i * _K + j, lbl[i * _K + j], 0, 0),
        )

    grid_spec = pltpu.PrefetchScalarGridSpec(
        num_scalar_prefetch=1,
        grid=(_STEPS,),
        in_specs=[_logit_spec(j) for j in range(_K)]
        + [pl.BlockSpec((_K, _M, _M), lambda i, lbl: (i, 0, 0))],
        out_specs=pl.BlockSpec(memory_space=pltpu.SMEM),
    )
    out = pl.pallas_call(
        _body,
        grid_spec=grid_spec,
        out_shape=jax.ShapeDtypeStruct((1, 1), jnp.float32),
    )(labels_i32, *([mask_logits] * _K), mask_targets)
    return out[0, 0]


# speedup vs baseline: 1.0441x; 1.0441x over previous
import jax
import jax.numpy as jnp
from jax import lax
from jax.experimental import pallas as pl
from jax.experimental.pallas import tpu as pltpu

_N, _C, _M = 1024, 81, 28
_CH = 128
_STEPS = _N // _CH


def _issue(labels_ref, logits_hbm, buf, sem, step, slot):
    def body(j, carry):
        p = step * _CH + j
        lbl = labels_ref[p]
        pltpu.make_async_copy(
            logits_hbm.at[p, lbl], buf.at[slot, j], sem
        ).start()
        return carry

    lax.fori_loop(0, _CH, body, 0)


def _wait(logits_hbm, buf, sem, slot):
    def body(j, carry):
        pltpu.make_async_copy(
            logits_hbm.at[0, 0], buf.at[slot, j], sem
        ).wait()
        return carry

    lax.fori_loop(0, _CH, body, 0)


def _body(labels_ref, logits_hbm, targets_ref, out_ref, buf, sem):
    i = pl.program_id(0)
    slot = lax.rem(i, 2)
    nslot = lax.rem(i + 1, 2)

    @pl.when(i == 0)
    def _():
        _issue(labels_ref, logits_hbm, buf, sem, 0, 0)

    @pl.when(i + 1 < _STEPS)
    def _():
        _issue(labels_ref, logits_hbm, buf, sem, i + 1, nslot)

    _wait(logits_hbm, buf, sem, slot)
    x = buf[slot]
    t = targets_ref[...]
    s = jnp.sum(jnp.maximum(x, 0.0) - x * t + jnp.log1p(jnp.exp(-jnp.abs(x))))
    prev = jnp.where(i == 0, 0.0, out_ref[0, 0])
    tot = prev + s
    out_ref[0, 0] = jnp.where(i == _STEPS - 1, tot / (_N * _M * _M), tot)


def kernel(mask_logits, labels, mask_targets):
    labels_i32 = labels.astype(jnp.int32)
    grid_spec = pltpu.PrefetchScalarGridSpec(
        num_scalar_prefetch=1,
        grid=(_STEPS,),
        in_specs=[
            pl.BlockSpec(memory_space=pl.ANY),
            pl.BlockSpec((_CH, _M, _M), lambda i, lbl: (i, 0, 0)),
        ],
        out_specs=pl.BlockSpec(memory_space=pltpu.SMEM),
        scratch_shapes=[
            pltpu.VMEM((2, _CH, _M, _M), jnp.float32),
            pltpu.SemaphoreType.DMA,
        ],
    )
    out = pl.pallas_call(
        _body,
        grid_spec=grid_spec,
        out_shape=jax.ShapeDtypeStruct((1, 1), jnp.float32),
    )(labels_i32, mask_logits, mask_targets)
    return out[0, 0]


# speedup vs baseline: 12.0479x; 11.5395x over previous
import jax
import jax.numpy as jnp
from jax import lax
from jax.experimental import pallas as pl
from jax.experimental.pallas import tpu as pltpu

_N, _C, _M = 1024, 81, 28
_P = _M * _M
_STEPS = _M


def _body(labels_ref, x_ref, t_ref, out_ref):
    i = pl.program_id(0)
    lbl = labels_ref[...]
    ci = lax.broadcasted_iota(jnp.int32, (_C, _N), 0)
    oh = (ci == lbl).astype(jnp.float32)
    x = x_ref[0]
    sel = jnp.sum(x * oh[None], axis=1)
    t = t_ref[0]
    bce = jnp.maximum(sel, 0.0) - sel * t + jnp.log1p(jnp.exp(-jnp.abs(sel)))
    s = jnp.sum(bce)
    prev = jnp.where(i == 0, 0.0, out_ref[0, 0])
    tot = prev + s
    out_ref[0, 0] = jnp.where(i == _STEPS - 1, tot / (_N * _P), tot)


def kernel(mask_logits, labels, mask_targets):
    xt = jnp.transpose(mask_logits, (2, 3, 1, 0))
    tt = jnp.transpose(mask_targets, (1, 2, 0))
    labels2 = labels.astype(jnp.int32).reshape(1, _N)
    grid_spec = pltpu.PrefetchScalarGridSpec(
        num_scalar_prefetch=0,
        grid=(_STEPS,),
        in_specs=[
            pl.BlockSpec((1, _N), lambda i: (0, 0)),
            pl.BlockSpec((1, _M, _C, _N), lambda i: (i, 0, 0, 0)),
            pl.BlockSpec((1, _M, _N), lambda i: (i, 0, 0)),
        ],
        out_specs=pl.BlockSpec(memory_space=pltpu.SMEM),
    )
    out = pl.pallas_call(
        _body,
        grid_spec=grid_spec,
        out_shape=jax.ShapeDtypeStruct((1, 1), jnp.float32),
    )(labels2, xt, tt)
    return out[0, 0]
